# baseline (speedup 1.0000x reference)
import functools

import jax
import jax.numpy as jnp
from jax import lax
from jax.experimental import pallas as pl
from jax.experimental.pallas import tpu as pltpu
from jax.experimental.pallas import tpu_sc as plsc

B, C, N = 8, 64, 16384
P, S = 2048, 32
NW = 32
WPB = NW // B
CPW = C // WPB
CBLK = 4
NSWEEP = CPW // CBLK
PCH = 128
NCH = P // PCH
T = NSWEEP * NCH

_mesh = plsc.VectorSubcoreMesh(core_axis_name="c", subcore_axis_name="s")


@functools.partial(
    pl.kernel,
    mesh=_mesh,
    out_type=jax.ShapeDtypeStruct((B, C, S, P), jnp.float32),
    scratch_types=[
        pltpu.VMEM((CBLK, N), jnp.float32),
        pltpu.VMEM((2, PCH, S), jnp.int32),
        pltpu.VMEM((2, CBLK, S, PCH), jnp.float32),
        pltpu.SemaphoreType.DMA((2,)),
        pltpu.SemaphoreType.DMA((2,)),
        pltpu.SemaphoreType.DMA,
    ],
    compiler_params=pltpu.CompilerParams(needs_layout_passes=False),
)
def _group_sc(feat_hbm, idx_hbm, out_hbm, feat_v, idx_v, out_v,
              isem, osem, fsem):
    cid = lax.axis_index("c")
    sid = lax.axis_index("s")
    w = sid * 2 + cid
    b = w // WPB
    c0 = (w % WPB) * CPW

    def idx_copy(t, buf):
        p0 = lax.rem(t, NCH) * PCH
        return pltpu.make_async_copy(
            idx_hbm.at[b, pl.ds(p0, PCH), :], idx_v.at[buf], isem.at[buf])

    def out_copy(t, buf):
        cbase = c0 + (t // NCH) * CBLK
        p0 = lax.rem(t, NCH) * PCH
        return pltpu.make_async_copy(
            out_v.at[buf],
            out_hbm.at[b, pl.ds(cbase, CBLK), :, pl.ds(p0, PCH)],
            osem.at[buf])

    def do_chunk(tp, t, buf):
        idx_copy(t, buf).wait()
        @pl.when(t + 1 < T)
        def _():
            idx_copy(t + 1, 1 - buf).start()
        @pl.when(tp > 0)
        def _():
            out_copy(t - 2, buf).wait()

        p_iota = lax.iota(jnp.int32, 16)
        bufv = jnp.full((16,), buf, jnp.int32)

        def pbody(pg, _):
            pbase = pg * 16
            pv = p_iota + pbase
            for s in range(S):
                iv = plsc.load_gather(
                    idx_v, [bufv, pv, jnp.full((16,), s, jnp.int32)])
                for cc in range(CBLK):
                    out_v[buf, cc, s, pl.ds(pbase, 16)] = plsc.load_gather(
                        feat_v, [jnp.full((16,), cc, jnp.int32), iv])
            return 0

        lax.fori_loop(0, PCH // 16, pbody, 0)
        out_copy(t, buf).start()

    def feat_copy(sweep):
        cbase = c0 + sweep * CBLK
        return pltpu.make_async_copy(
            feat_hbm.at[b, pl.ds(cbase, CBLK), :], feat_v, fsem)

    idx_copy(0, 0).start()

    def pair(tp, _):
        @pl.when(lax.rem(tp, T // (2 * NSWEEP)) == 0)
        def _():
            fc = feat_copy(tp // (T // (2 * NSWEEP)))
            fc.start()
            fc.wait()

        do_chunk(tp, 2 * tp, 0)
        do_chunk(tp, 2 * tp + 1, 1)
        return 0

    lax.fori_loop(0, T // 2, pair, 0)

    out_copy(T - 2, 0).wait()
    out_copy(T - 1, 1).wait()


def kernel(features, idx):
    idx32 = idx.astype(jnp.int32)
    out = _group_sc(features, idx32)
    return jnp.transpose(out, (0, 1, 3, 2))


# speedup vs baseline: 2.8405x; 2.8405x over previous
import functools

import jax
import jax.numpy as jnp
from jax import lax
from jax.experimental import pallas as pl
from jax.experimental.pallas import tpu as pltpu
from jax.experimental.pallas import tpu_sc as plsc

B, C, N = 8, 64, 16384
P, S = 2048, 32
NW = 32
WPB = NW // B
CPW = C // WPB
CBLK = 4
NSWEEP = CPW // CBLK
PCH = 128
NCH = P // PCH
T = NSWEEP * NCH

_mesh = plsc.VectorSubcoreMesh(core_axis_name="c", subcore_axis_name="s")


@functools.partial(
    pl.kernel,
    mesh=_mesh,
    out_type=jax.ShapeDtypeStruct((B, C, S, P), jnp.float32),
    scratch_types=[
        pltpu.VMEM((CBLK, N), jnp.float32),
        pltpu.VMEM((2, PCH, S), jnp.int32),
        pltpu.VMEM((2, CBLK, S, PCH), jnp.float32),
        pltpu.SemaphoreType.DMA((2,)),
        pltpu.SemaphoreType.DMA((2,)),
        pltpu.SemaphoreType.DMA,
    ],
    compiler_params=pltpu.CompilerParams(needs_layout_passes=False),
)
def _group_sc(feat_hbm, idx_hbm, out_hbm, feat_v, idx_v, out_v,
              isem, osem, fsem):
    cid = lax.axis_index("c")
    sid = lax.axis_index("s")
    w = sid * 2 + cid
    b = w // WPB
    c0 = (w % WPB) * CPW

    def idx_copy(t, buf):
        p0 = lax.rem(t, NCH) * PCH
        return pltpu.make_async_copy(
            idx_hbm.at[b, pl.ds(p0, PCH), :], idx_v.at[buf], isem.at[buf])

    def out_copy(t, buf):
        cbase = c0 + (t // NCH) * CBLK
        p0 = lax.rem(t, NCH) * PCH
        return pltpu.make_async_copy(
            out_v.at[buf],
            out_hbm.at[b, pl.ds(cbase, CBLK), :, pl.ds(p0, PCH)],
            osem.at[buf])

    def do_chunk(tp, t, buf):
        idx_copy(t, buf).wait()
        @pl.when(t + 1 < T)
        def _():
            idx_copy(t + 1, 1 - buf).start()
        @pl.when(tp > 0)
        def _():
            out_copy(t - 2, buf).wait()

        p_iota = lax.iota(jnp.int32, 16)
        bufv = jnp.full((16,), buf, jnp.int32)
        ccv = [jnp.full((16,), cc, jnp.int32) for cc in range(CBLK)]

        @plsc.parallel_loop(0, (PCH // 16) * S, unroll=4)
        def _gather(i):
            pg = lax.shift_right_logical(i, 5)
            s = lax.bitwise_and(i, S - 1)
            pbase = pg * 16
            pv = p_iota + pbase
            iv = plsc.load_gather(
                idx_v, [bufv, pv, jnp.full((16,), s, jnp.int32)])
            for cc in range(CBLK):
                out_v[buf, cc, s, pl.ds(pbase, 16)] = plsc.load_gather(
                    feat_v, [ccv[cc], iv])

        out_copy(t, buf).start()

    def feat_copy(sweep):
        cbase = c0 + sweep * CBLK
        return pltpu.make_async_copy(
            feat_hbm.at[b, pl.ds(cbase, CBLK), :], feat_v, fsem)

    idx_copy(0, 0).start()

    def pair(tp, _):
        @pl.when(lax.rem(tp, T // (2 * NSWEEP)) == 0)
        def _():
            fc = feat_copy(tp // (T // (2 * NSWEEP)))
            fc.start()
            fc.wait()

        do_chunk(tp, 2 * tp, 0)
        do_chunk(tp, 2 * tp + 1, 1)
        return 0

    lax.fori_loop(0, T // 2, pair, 0)

    out_copy(T - 2, 0).wait()
    out_copy(T - 1, 1).wait()


def kernel(features, idx):
    idx32 = idx.astype(jnp.int32)
    out = _group_sc(features, idx32)
    return jnp.transpose(out, (0, 1, 3, 2))


# speedup vs baseline: 5.0389x; 1.7739x over previous
import functools

import jax
import jax.numpy as jnp
from jax import lax
from jax.experimental import pallas as pl
from jax.experimental.pallas import tpu as pltpu
from jax.experimental.pallas import tpu_sc as plsc

B, C, N = 8, 64, 16384
P, S = 2048, 32
NW = 32
WPB = NW // B
CPW = C // WPB
CBLK = 4
NSWEEP = CPW // CBLK
PCH = 128
NCH = P // PCH
T = NSWEEP * NCH

_mesh = plsc.VectorSubcoreMesh(core_axis_name="c", subcore_axis_name="s")


@functools.partial(
    pl.kernel,
    mesh=_mesh,
    out_type=jax.ShapeDtypeStruct((B, C, S, P), jnp.float32),
    scratch_types=[
        pltpu.VMEM((CBLK, N), jnp.float32),
        pltpu.VMEM((2, S, PCH), jnp.int32),
        pltpu.VMEM((2, CBLK, S, PCH), jnp.float32),
        pltpu.SemaphoreType.DMA((2,)),
        pltpu.SemaphoreType.DMA((2,)),
        pltpu.SemaphoreType.DMA,
    ],
    compiler_params=pltpu.CompilerParams(needs_layout_passes=False),
)
def _group_sc(feat_hbm, idx_hbm, out_hbm, feat_v, idx_v, out_v,
              isem, osem, fsem):
    cid = lax.axis_index("c")
    sid = lax.axis_index("s")
    w = sid * 2 + cid
    b = w // WPB
    c0 = (w % WPB) * CPW

    def idx_copy(t, buf):
        p0 = lax.rem(t, NCH) * PCH
        return pltpu.make_async_copy(
            idx_hbm.at[b, :, pl.ds(p0, PCH)], idx_v.at[buf], isem.at[buf])

    def out_copy(t, buf):
        cbase = c0 + (t // NCH) * CBLK
        p0 = lax.rem(t, NCH) * PCH
        return pltpu.make_async_copy(
            out_v.at[buf],
            out_hbm.at[b, pl.ds(cbase, CBLK), :, pl.ds(p0, PCH)],
            osem.at[buf])

    def do_chunk(tp, t, buf):
        idx_copy(t, buf).wait()
        @pl.when(t + 1 < T)
        def _():
            idx_copy(t + 1, 1 - buf).start()
        @pl.when(tp > 0)
        def _():
            out_copy(t - 2, buf).wait()

        ccv = [jnp.full((16,), cc, jnp.int32) for cc in range(CBLK)]

        @plsc.parallel_loop(0, (PCH // 16) * S, unroll=4)
        def _gather(i):
            pg = lax.shift_right_logical(i, 5)
            s = lax.bitwise_and(i, S - 1)
            pbase = pg * 16
            iv = idx_v[buf, s, pl.ds(pbase, 16)]
            for cc in range(CBLK):
                out_v[buf, cc, s, pl.ds(pbase, 16)] = plsc.load_gather(
                    feat_v, [ccv[cc], iv])

        out_copy(t, buf).start()

    def feat_copy(sweep):
        cbase = c0 + sweep * CBLK
        return pltpu.make_async_copy(
            feat_hbm.at[b, pl.ds(cbase, CBLK), :], feat_v, fsem)

    idx_copy(0, 0).start()

    def pair(tp, _):
        @pl.when(lax.rem(tp, T // (2 * NSWEEP)) == 0)
        def _():
            fc = feat_copy(tp // (T // (2 * NSWEEP)))
            fc.start()
            fc.wait()

        do_chunk(tp, 2 * tp, 0)
        do_chunk(tp, 2 * tp + 1, 1)
        return 0

    lax.fori_loop(0, T // 2, pair, 0)

    out_copy(T - 2, 0).wait()
    out_copy(T - 1, 1).wait()


def kernel(features, idx):
    idx_t = jnp.transpose(idx.astype(jnp.int32), (0, 2, 1))
    out = _group_sc(features, idx_t)
    return jnp.transpose(out, (0, 1, 3, 2))


# speedup vs baseline: 5.0530x; 1.0028x over previous
import functools

import jax
import jax.numpy as jnp
from jax import lax
from jax.experimental import pallas as pl
from jax.experimental.pallas import tpu as pltpu
from jax.experimental.pallas import tpu_sc as plsc

B, C, N = 8, 64, 16384
P, S = 2048, 32
NW = 32
WPB = NW // B
CPW = C // WPB
CBLK = 4
NSWEEP = CPW // CBLK
PCH = 128
NCH = P // PCH
T = NSWEEP * NCH

_mesh = plsc.VectorSubcoreMesh(core_axis_name="c", subcore_axis_name="s")


@functools.partial(
    pl.kernel,
    mesh=_mesh,
    out_type=jax.ShapeDtypeStruct((B, C, S, P), jnp.float32),
    scratch_types=[
        pltpu.VMEM((CBLK, N), jnp.float32),
        pltpu.VMEM((2, S, PCH), jnp.int32),
        pltpu.VMEM((2, CBLK, S, PCH), jnp.float32),
        pltpu.SemaphoreType.DMA((2,)),
        pltpu.SemaphoreType.DMA((2,)),
        pltpu.SemaphoreType.DMA,
    ],
    compiler_params=pltpu.CompilerParams(needs_layout_passes=False),
)
def _group_sc(feat_hbm, idx_hbm, out_hbm, feat_v, idx_v, out_v,
              isem, osem, fsem):
    cid = lax.axis_index("c")
    sid = lax.axis_index("s")
    w = sid * 2 + cid
    b = w // WPB
    c0 = (w % WPB) * CPW

    def idx_copy(t, buf):
        p0 = lax.rem(t, NCH) * PCH
        return pltpu.make_async_copy(
            idx_hbm.at[b, :, pl.ds(p0, PCH)], idx_v.at[buf], isem.at[buf])

    def out_copy(t, buf):
        cbase = c0 + (t // NCH) * CBLK
        p0 = lax.rem(t, NCH) * PCH
        return pltpu.make_async_copy(
            out_v.at[buf],
            out_hbm.at[b, pl.ds(cbase, CBLK), :, pl.ds(p0, PCH)],
            osem.at[buf])

    def do_chunk(tp, t, buf):
        idx_copy(t, buf).wait()
        @pl.when(t + 1 < T)
        def _():
            idx_copy(t + 1, 1 - buf).start()
        @pl.when(tp > 0)
        def _():
            out_copy(t - 2, buf).wait()

        ccv = [jnp.full((16,), cc, jnp.int32) for cc in range(CBLK)]

        @plsc.parallel_loop(0, (PCH // 16) * S, unroll=8)
        def _gather(i):
            pg = lax.shift_right_logical(i, 5)
            s = lax.bitwise_and(i, S - 1)
            pbase = pg * 16
            iv = idx_v[buf, s, pl.ds(pbase, 16)]
            for cc in range(CBLK):
                out_v[buf, cc, s, pl.ds(pbase, 16)] = plsc.load_gather(
                    feat_v, [ccv[cc], iv])

        out_copy(t, buf).start()

    def feat_copy(sweep):
        cbase = c0 + sweep * CBLK
        return pltpu.make_async_copy(
            feat_hbm.at[b, pl.ds(cbase, CBLK), :], feat_v, fsem)

    idx_copy(0, 0).start()

    def pair(tp, _):
        @pl.when(lax.rem(tp, T // (2 * NSWEEP)) == 0)
        def _():
            fc = feat_copy(tp // (T // (2 * NSWEEP)))
            fc.start()
            fc.wait()

        do_chunk(tp, 2 * tp, 0)
        do_chunk(tp, 2 * tp + 1, 1)
        return 0

    lax.fori_loop(0, T // 2, pair, 0)

    out_copy(T - 2, 0).wait()
    out_copy(T - 1, 1).wait()


def kernel(features, idx):
    idx_t = jnp.transpose(idx.astype(jnp.int32), (0, 2, 1))
    out = _group_sc(features, idx_t)
    return jnp.transpose(out, (0, 1, 3, 2))
